# Initial kernel scaffold; baseline (speedup 1.0000x reference)
import jax
import jax.numpy as jnp
from jax.experimental import pallas as pl


def kernel(A, W0, b0, W1, b1, W2, b2, W3, b3, p1, p2, p3, U0, ub0, U1, ub1, U2, ub2, Wup, bup, Wc1, bc1, Wc2, bc2):
    raise NotImplementedError("write your pallas kernel here")


# speedup vs baseline: 2.9945x; 2.9945x over previous
import jax
import jax.numpy as jnp
from jax.experimental import pallas as pl

F32 = jnp.float32
HID = 256


def _dot(a, b):
    return jax.lax.dot_general(a, b, (((1,), (0,)), ((), ())),
                               preferred_element_type=F32)


def _dotT(a, b):
    return jax.lax.dot_general(a, b, (((0,), (0,)), ((), ())),
                               preferred_element_type=F32)


def _dotBT(a, b):
    return jax.lax.dot_general(a, b, (((1,), (1,)), ((), ())),
                               preferred_element_type=F32)


def _rowsum(a):
    return jnp.sum(a, axis=1, keepdims=True)


def _eye(m):
    r = jax.lax.broadcasted_iota(jnp.int32, (m, m), 0)
    c = jax.lax.broadcasted_iota(jnp.int32, (m, m), 1)
    return (r == c).astype(F32)


def _diag_col(a):
    return _rowsum(a * _eye(a.shape[0]))


def _gcn_generic(X, A, d, W, b_row, fill):
    dfix = jnp.where(d == 0.0, fill, d)
    deg = _rowsum(A) - d + dfix
    deg_safe = jnp.where(deg > 0.0, deg, 1.0)
    dinv = jnp.where(deg > 0.0, jax.lax.rsqrt(deg_safe), 0.0)
    Y = dinv * _dot(X, W)
    return dinv * (_dot(A, Y) + (dfix - d) * Y) + b_row


def _gcn_zerodiag(X, Ap, W, b_row, fill):
    deg = _rowsum(Ap) + fill
    dinv = jax.lax.rsqrt(deg)
    Y = dinv * _dot(X, W)
    return dinv * (_dot(Ap, Y) + fill * Y) + b_row


def _pool_mats(X, p_col):
    m = X.shape[0]
    k = m // 2
    norm = jnp.sqrt(jnp.sum(p_col * p_col))
    s = jnp.tanh(_dot(X, p_col) / norm)
    sT = jnp.transpose(s)
    ri = jax.lax.broadcasted_iota(jnp.int32, (m, m), 0)
    ci = jax.lax.broadcasted_iota(jnp.int32, (m, m), 1)
    beats = (sT > s) | ((sT == s) & (ci < ri))
    rank = jnp.sum(beats.astype(jnp.int32), axis=1, keepdims=True)
    kio = jax.lax.broadcasted_iota(jnp.int32, (m, k), 1)
    PT = (rank == kio).astype(F32)
    vals = _dotT(PT, s)
    return PT, vals


def _down_level(X, A_cur, p_col, W, b_row):
    m = A_cur.shape[0]
    k = m // 2
    eye = _eye(m)
    B = A_cur * (1.0 - eye) + eye
    PT, vals = _pool_mats(X, p_col)
    PB = _dotT(PT, B)
    BP = _dot(B, PT)
    Ap = _dot(PB, BP) * (1.0 - _eye(k))
    Xp = _dotT(PT, X) * vals
    Xn = jax.nn.relu(_gcn_zerodiag(Xp, Ap, W, b_row, 2.0))
    return Xn, Ap, PT


def _k1(A_ref, W0, b0, W1, b1, W2, b2, W3, b3, p1, p2, p3,
        U0, ub0, U1, ub1, U2, ub2, X_out):
    A = A_ref[...]
    d = _diag_col(A)
    dfix = jnp.where(d == 0.0, 2.0, d)
    deg = _rowsum(A) - d + dfix
    deg_safe = jnp.where(deg > 0.0, deg, 1.0)
    dinv = jnp.where(deg > 0.0, jax.lax.rsqrt(deg_safe), 0.0)
    s_row = jnp.sum(W0[...], axis=0, keepdims=True)
    q = dinv * (_dot(A, dinv) + (dfix - d) * dinv)
    X0 = jax.nn.relu(q * s_row + b0[...])

    X1, Ap1, PT1 = _down_level(X0, A, p1[...], W1[...], b1[...])
    X2, Ap2, PT2 = _down_level(X1, Ap1, p2[...], W2[...], b2[...])
    X3, _, PT3 = _down_level(X2, Ap2, p3[...], W3[...], b3[...])

    X = jax.nn.relu(_gcn_zerodiag(X2 + _dot(PT3, X3), Ap2, U0[...], ub0[...], 2.0))
    X = jax.nn.relu(_gcn_zerodiag(X1 + _dot(PT2, X), Ap1, U1[...], ub1[...], 2.0))
    X_out[...] = _gcn_generic(X0 + _dot(PT1, X), A, d, U2[...], ub2[...], 2.0)


def _k2(A_ref, X_ref, Wup, bup, Xu_out, Au_out, dau_out):
    n = A_ref.shape[0]
    A = A_ref[...]
    X = X_ref[...]
    nn = _dot(Wup[...], X) + bup[...]
    Xu_out[:n, :] = X
    Xu_out[n:, :] = nn
    Au_out[:n, :n] = A
    Au_out[n:, :n] = A
    Au_out[:n, n:] = jnp.transpose(A)
    Au_out[n:, n:] = jax.nn.sigmoid(_dotBT(nn, nn))
    dau_out[:n, :] = _diag_col(A)
    dau_out[n:, :] = jax.nn.sigmoid(_rowsum(nn * nn))


def _k3(Xu_ref, Ae_ref, dae_ref, Wc1, bc1, Wc2, bc2,
        Xu_out, Am_out, dm_out):
    X1 = jax.nn.relu(
        _gcn_generic(Xu_ref[...], Ae_ref[...], dae_ref[...],
                     Wc1[...], bc1[...], 1.0))
    Am = jax.nn.sigmoid(_dotBT(X1, X1))
    Am_out[...] = Am
    dm = jax.nn.sigmoid(_rowsum(X1 * X1))
    dm_out[...] = dm
    Xu_out[...] = jax.nn.relu(
        _gcn_generic(X1, Am, dm, Wc2[...], bc2[...], 1.0))


def _k4(Xu_ref, out):
    X = Xu_ref[...]
    out[...] = jax.nn.sigmoid(_dotBT(X, X))


def kernel(A, W0, b0, W1, b1, W2, b2, W3, b3, p1, p2, p3,
           U0, ub0, U1, ub1, U2, ub2, Wup, bup, Wc1, bc1, Wc2, bc2):
    n = A.shape[0]
    m = 2 * n
    b0r = b0.reshape(1, -1)
    b1r = b1.reshape(1, -1)
    b2r = b2.reshape(1, -1)
    b3r = b3.reshape(1, -1)
    ub0r = ub0.reshape(1, -1)
    ub1r = ub1.reshape(1, -1)
    ub2r = ub2.reshape(1, -1)
    bc1r = bc1.reshape(1, -1)
    bc2r = bc2.reshape(1, -1)
    p1c = p1.reshape(-1, 1)
    p2c = p2.reshape(-1, 1)
    p3c = p3.reshape(-1, 1)
    bupc = bup.reshape(-1, 1)

    X = pl.pallas_call(
        _k1,
        out_shape=jax.ShapeDtypeStruct((n, HID), F32),
    )(A, W0, b0r, W1, b1r, W2, b2r, W3, b3r, p1c, p2c, p3c,
      U0, ub0r, U1, ub1r, U2, ub2r)

    Xu, Ae, dae = pl.pallas_call(
        _k2,
        out_shape=(
            jax.ShapeDtypeStruct((m, HID), F32),
            jax.ShapeDtypeStruct((m, m), F32),
            jax.ShapeDtypeStruct((m, 1), F32),
        ),
    )(A, X, Wup, bupc)

    refine = pl.pallas_call(
        _k3,
        out_shape=(
            jax.ShapeDtypeStruct((m, HID), F32),
            jax.ShapeDtypeStruct((m, m), F32),
            jax.ShapeDtypeStruct((m, 1), F32),
        ),
    )
    for _ in range(3):
        Xu, Ae, dae = refine(Xu, Ae, dae, Wc1, bc1r, Wc2, bc2r)

    return pl.pallas_call(
        _k4,
        out_shape=jax.ShapeDtypeStruct((m, m), F32),
    )(Xu)
